# baseline (speedup 1.0000x reference)
import functools

import jax
import jax.numpy as jnp
import numpy as np
from jax import lax
from jax.experimental import pallas as pl
from jax.experimental.pallas import tpu as pltpu
from jax.experimental.pallas import tpu_sc as plsc

N, F, H, E = 10000, 128, 128, 320000
NC, NS = 2, 16
NW = NC * NS
CHUNK = 80
K0 = 98
K1 = 154
PK = 2 * CHUNK
EPAD = NS * (K0 + K1) * CHUNK
NP = 10240
ROWS_PS = NP // NS
LANES = 16
INV_SQRT_H = float(1.0 / np.sqrt(H))


_GDN = lax.GatherDimensionNumbers(offset_dims=(), collapsed_slice_dims=(0,),
                                  start_index_map=(0,))


def _bcast_lane(vec, lane):
    idx = jnp.full((LANES, 1), lane, jnp.int32)
    return lax.gather(vec, idx, _GDN, slice_sizes=(1,),
                      mode=lax.GatherScatterMode.PROMISE_IN_BOUNDS)

def _edge_body(m_hbm, pk_hbm, len_hbm, zeros_hbm, out_hbm,
               agg, i0, i1, w0, w1, d0, d1, g0, g1, s0, s1,
               im0, im1, wm0, wm1, gm0, gm1, sm0, sm1):
    ibufs, wbufs, dbufs = (i0, i1), (w0, w1), (d0, d1)
    gbufs, sbufs = (g0, g1), (s0, s1)
    isems, wsems = (im0, im1), (wm0, wm1)
    gsems, ssems = (gm0, gm1), (sm0, sm1)
    c = lax.axis_index("c")
    sid = lax.axis_index("s")
    base = jnp.where(c == 0, sid * K0, NS * K0 + sid * K1)
    nch = jnp.where(c == 0, K0, K1)

    def idx_cp(j, b):
        return pltpu.make_async_copy(
            pk_hbm.at[pl.ds((base + j) * PK, PK)], ibufs[b], isems[b])

    def len_cp(j, b):
        return pltpu.make_async_copy(
            len_hbm.at[pl.ds((base + j) * CHUNK, CHUNK)], wbufs[b],
            wsems[b])

    def gather_cp(b):
        return pltpu.make_async_copy(
            m_hbm.at[ibufs[b].at[pl.ds(0, CHUNK)]], gbufs[b], gsems[b])

    def scatter_issue(b):
        pltpu.async_copy(sbufs[b], agg.at[dbufs[b].at[0]], ssems[b],
                         add=True)

    def scatter_wait(b):
        pltpu.make_async_copy(sbufs[b], agg.at[dbufs[b].at[0]],
                              ssems[b]).wait()

    def copy_dst(b):
        for q in range(CHUNK // LANES):
            sl = pl.ds(q * LANES, LANES)
            dbufs[b][0, sl] = ibufs[b][pl.ds(CHUNK + q * LANES, LANES)]

    def scale(b):
        def _group(gi, inner):
            wvec = 1.0 / (1.0 + wbufs[b][pl.ds(gi * LANES, LANES)])
            for l in range(LANES):
                wb = _bcast_lane(wvec, l)
                e = gi * LANES + l
                for r in range(H // LANES):
                    sl = pl.ds(r * LANES, LANES)
                    sbufs[b][e, sl] = gbufs[b][e, sl] * wb
            return inner

        lax.fori_loop(0, CHUNK // LANES, _group, 0)

    pltpu.sync_copy(zeros_hbm, agg.at[pl.ds(sid * ROWS_PS, ROWS_PS)])
    idx_cp(0, 0).start()
    len_cp(0, 0).start()
    idx_cp(1, 1).start()
    len_cp(1, 1).start()
    idx_cp(0, 0).wait()
    len_cp(0, 0).wait()
    gather_cp(0).start()
    plsc.subcore_barrier()

    def step(j, b, first, last):
        if not last:
            idx_cp(j + 1, 1 - b).wait()
            len_cp(j + 1, 1 - b).wait()
            gather_cp(1 - b).start()
        gather_cp(b).wait()
        if not first:
            scatter_wait(b)
        copy_dst(b)
        scale(b)
        scatter_issue(b)
        if not (first or last):
            idx_cp(j + 2, b).start()
            len_cp(j + 2, b).start()

    step(0, 0, True, False)
    idx_cp(2, 0).start()
    len_cp(2, 0).start()
    step(1, 1, True, False)
    idx_cp(3, 1).start()
    len_cp(3, 1).start()

    def _steady(o, carry):
        step(2 * o, 0, False, False)
        step(2 * o + 1, 1, False, False)
        return carry

    lax.fori_loop(1, nch // 2 - 1, _steady, 0)

    step(nch - 2, 0, False, False)
    step(nch - 1, 1, False, True)
    scatter_wait(0)
    scatter_wait(1)
    idx_cp(nch, 0).wait()
    len_cp(nch, 0).wait()

    plsc.subcore_barrier()

    pltpu.sync_copy(agg.at[pl.ds(sid * ROWS_PS, ROWS_PS)],
                    out_hbm.at[c, pl.ds(sid * ROWS_PS, ROWS_PS)])


@functools.partial(
    pl.kernel,
    out_type=jax.ShapeDtypeStruct((NC, NP, H), jnp.float32),
    mesh=plsc.VectorSubcoreMesh(core_axis_name="c", subcore_axis_name="s",
                                num_cores=NC, num_subcores=NS),
    scratch_types=[
        pltpu.VMEM_SHARED((NP, H), jnp.float32),
        pltpu.VMEM((PK,), jnp.int32),
        pltpu.VMEM((PK,), jnp.int32),
        pltpu.VMEM((CHUNK,), jnp.float32),
        pltpu.VMEM((CHUNK,), jnp.float32),
        pltpu.VMEM((1, CHUNK), jnp.int32),
        pltpu.VMEM((1, CHUNK), jnp.int32),
        pltpu.VMEM((CHUNK, H), jnp.float32),
        pltpu.VMEM((CHUNK, H), jnp.float32),
        pltpu.VMEM((CHUNK, H), jnp.float32),
        pltpu.VMEM((CHUNK, H), jnp.float32),
    ] + [pltpu.SemaphoreType.DMA] * 8,
)
def _edge_kernel(*args):
    _edge_body(*args)


def _proj_body(x_ref, w_ref, b_ref, ak_ref, h_ref, m_ref):
    x = x_ref[...]
    h = lax.dot_general(x, w_ref[...], (((1,), (1,)), ((), ())),
                        preferred_element_type=jnp.float32) + b_ref[...]
    h_ref[...] = h
    key = jnp.exp(jnp.sum(h * ak_ref[...], axis=1, keepdims=True)
                  * INV_SQRT_H)
    m_ref[...] = h * key


def _layer_body(last, h_ref, p0_ref, p1_ref, w_ref, b_ref, g_ref, be_ref,
                ak_ref, *out_refs):
    agg = jnp.maximum(p0_ref[...] + p1_ref[...], 0.0)
    h = h_ref[...] + lax.dot_general(
        agg, w_ref[...], (((1,), (1,)), ((), ())),
        preferred_element_type=jnp.float32) + b_ref[...]
    mean = jnp.mean(h, axis=0, keepdims=True)
    d = h - mean
    var = jnp.mean(d * d, axis=0, keepdims=True)
    hn = d * lax.rsqrt(var + 1e-5) * g_ref[...] + be_ref[...]
    out_refs[0][...] = hn
    if not last:
        key = jnp.exp(jnp.sum(hn * ak_ref[...], axis=1, keepdims=True)
                      * INV_SQRT_H)
        out_refs[1][...] = hn * key


_f32 = lambda *shape: jax.ShapeDtypeStruct(shape, jnp.float32)

_proj_call = pl.pallas_call(_proj_body, out_shape=[_f32(N, H), _f32(N, H)])
_mid_call = pl.pallas_call(functools.partial(_layer_body, False),
                           out_shape=[_f32(N, H), _f32(N, H)])
_last_call = pl.pallas_call(functools.partial(_layer_body, True),
                            out_shape=[_f32(N, H)])


def kernel(node_init, edge_index, edge_len, W_in, b_in, W1, b1, W2, b2,
           g1, be1, g2, be2, att_k):
    pad = EPAD - E
    srcp = jnp.pad(edge_index[0], (0, pad)).reshape(-1, CHUNK)
    dstp = jnp.pad(edge_index[1], (0, pad)).reshape(-1, CHUNK)
    pk = jnp.pad(jnp.stack([srcp, dstp], axis=1).reshape(-1), (0, PK))
    lenf = jnp.pad(edge_len, (0, pad + CHUNK), constant_values=np.inf)
    zeros = jnp.zeros((ROWS_PS, H), jnp.float32)
    b_in2 = b_in.reshape(1, H)
    b1_2, b2_2 = b1.reshape(1, H), b2.reshape(1, H)
    g1_2, g2_2 = g1.reshape(1, H), g2.reshape(1, H)
    be1_2, be2_2 = be1.reshape(1, H), be2.reshape(1, H)
    ak2 = att_k.reshape(1, H)

    h0, m0 = _proj_call(node_init, W_in, b_in2, ak2)
    part = _edge_kernel(m0, pk, lenf, zeros)
    h1, m1 = _mid_call(h0, part[0, :N], part[1, :N], W1, b1_2, g1_2,
                       be1_2, ak2)
    part = _edge_kernel(m1, pk, lenf, zeros)
    h2 = _last_call(h1, part[0, :N], part[1, :N], W2, b2_2, g2_2, be2_2,
                    ak2)
    return h2[0] if isinstance(h2, (list, tuple)) else h2


# speedup vs baseline: 1.1689x; 1.1689x over previous
import functools

import jax
import jax.numpy as jnp
import numpy as np
from jax import lax
from jax.experimental import pallas as pl
from jax.experimental.pallas import tpu as pltpu
from jax.experimental.pallas import tpu_sc as plsc

N, F, H, E = 10000, 128, 128, 320000
NC, NS = 2, 16
NW = NC * NS
CHUNK = 80
K0 = 154
K1 = 98
PK = 2 * CHUNK
EPAD = NS * (K0 + K1) * CHUNK
NP = 10240
ROWS_PS = NP // NS
LANES = 16
INV_SQRT_H = float(1.0 / np.sqrt(H))


_GDN = lax.GatherDimensionNumbers(offset_dims=(), collapsed_slice_dims=(0,),
                                  start_index_map=(0,))


def _bcast_lane(vec, lane):
    idx = jnp.full((LANES, 1), lane, jnp.int32)
    return lax.gather(vec, idx, _GDN, slice_sizes=(1,),
                      mode=lax.GatherScatterMode.PROMISE_IN_BOUNDS)

def _edge_body(m_hbm, pk_hbm, len_hbm, zeros_hbm, out_hbm,
               agg, i0, i1, w0, w1, d0, d1, g0, g1, s0, s1,
               im0, im1, wm0, wm1, gm0, gm1, sm0, sm1):
    ibufs, wbufs, dbufs = (i0, i1), (w0, w1), (d0, d1)
    gbufs, sbufs = (g0, g1), (s0, s1)
    isems, wsems = (im0, im1), (wm0, wm1)
    gsems, ssems = (gm0, gm1), (sm0, sm1)
    c = lax.axis_index("c")
    sid = lax.axis_index("s")
    base = jnp.where(c == 0, sid * K0, NS * K0 + sid * K1)
    nch = jnp.where(c == 0, K0, K1)

    def idx_cp(j, b):
        return pltpu.make_async_copy(
            pk_hbm.at[pl.ds((base + j) * PK, PK)], ibufs[b], isems[b])

    def len_cp(j, b):
        return pltpu.make_async_copy(
            len_hbm.at[pl.ds((base + j) * CHUNK, CHUNK)], wbufs[b],
            wsems[b])

    def gather_cp(b):
        return pltpu.make_async_copy(
            m_hbm.at[ibufs[b].at[pl.ds(0, CHUNK)]], gbufs[b], gsems[b])

    def scatter_issue(b):
        pltpu.async_copy(sbufs[b], agg.at[dbufs[b].at[0]], ssems[b],
                         add=True)

    def scatter_wait(b):
        pltpu.make_async_copy(sbufs[b], agg.at[dbufs[b].at[0]],
                              ssems[b]).wait()

    def copy_dst(b):
        for q in range(CHUNK // LANES):
            sl = pl.ds(q * LANES, LANES)
            dbufs[b][0, sl] = ibufs[b][pl.ds(CHUNK + q * LANES, LANES)]

    def scale(b):
        def _group(gi, inner):
            wvec = 1.0 / (1.0 + wbufs[b][pl.ds(gi * LANES, LANES)])
            for l in range(LANES):
                wb = _bcast_lane(wvec, l)
                e = gi * LANES + l
                for r in range(H // LANES):
                    sl = pl.ds(r * LANES, LANES)
                    sbufs[b][e, sl] = gbufs[b][e, sl] * wb
            return inner

        lax.fori_loop(0, CHUNK // LANES, _group, 0)

    pltpu.sync_copy(zeros_hbm, agg.at[pl.ds(sid * ROWS_PS, ROWS_PS)])
    idx_cp(0, 0).start()
    len_cp(0, 0).start()
    idx_cp(1, 1).start()
    len_cp(1, 1).start()
    idx_cp(0, 0).wait()
    len_cp(0, 0).wait()
    gather_cp(0).start()
    plsc.subcore_barrier()

    def step(j, b, first, last):
        if not last:
            idx_cp(j + 1, 1 - b).wait()
            len_cp(j + 1, 1 - b).wait()
            gather_cp(1 - b).start()
        gather_cp(b).wait()
        if not first:
            scatter_wait(b)
        copy_dst(b)
        scale(b)
        scatter_issue(b)
        if not (first or last):
            idx_cp(j + 2, b).start()
            len_cp(j + 2, b).start()

    step(0, 0, True, False)
    idx_cp(2, 0).start()
    len_cp(2, 0).start()
    step(1, 1, True, False)
    idx_cp(3, 1).start()
    len_cp(3, 1).start()

    def _steady(o, carry):
        step(2 * o, 0, False, False)
        step(2 * o + 1, 1, False, False)
        return carry

    lax.fori_loop(1, nch // 2 - 1, _steady, 0)

    step(nch - 2, 0, False, False)
    step(nch - 1, 1, False, True)
    scatter_wait(0)
    scatter_wait(1)
    idx_cp(nch, 0).wait()
    len_cp(nch, 0).wait()

    plsc.subcore_barrier()

    pltpu.sync_copy(agg.at[pl.ds(sid * ROWS_PS, ROWS_PS)],
                    out_hbm.at[c, pl.ds(sid * ROWS_PS, ROWS_PS)])


@functools.partial(
    pl.kernel,
    out_type=jax.ShapeDtypeStruct((NC, NP, H), jnp.float32),
    mesh=plsc.VectorSubcoreMesh(core_axis_name="c", subcore_axis_name="s",
                                num_cores=NC, num_subcores=NS),
    scratch_types=[
        pltpu.VMEM_SHARED((NP, H), jnp.float32),
        pltpu.VMEM((PK,), jnp.int32),
        pltpu.VMEM((PK,), jnp.int32),
        pltpu.VMEM((CHUNK,), jnp.float32),
        pltpu.VMEM((CHUNK,), jnp.float32),
        pltpu.VMEM((1, CHUNK), jnp.int32),
        pltpu.VMEM((1, CHUNK), jnp.int32),
        pltpu.VMEM((CHUNK, H), jnp.float32),
        pltpu.VMEM((CHUNK, H), jnp.float32),
        pltpu.VMEM((CHUNK, H), jnp.float32),
        pltpu.VMEM((CHUNK, H), jnp.float32),
    ] + [pltpu.SemaphoreType.DMA] * 8,
)
def _edge_kernel(*args):
    _edge_body(*args)


def _proj_body(x_ref, w_ref, b_ref, ak_ref, h_ref, m_ref):
    x = x_ref[...]
    h = lax.dot_general(x, w_ref[...], (((1,), (1,)), ((), ())),
                        preferred_element_type=jnp.float32) + b_ref[...]
    h_ref[...] = h
    key = jnp.exp(jnp.sum(h * ak_ref[...], axis=1, keepdims=True)
                  * INV_SQRT_H)
    m_ref[...] = h * key


def _layer_body(last, h_ref, p0_ref, p1_ref, w_ref, b_ref, g_ref, be_ref,
                ak_ref, *out_refs):
    agg = jnp.maximum(p0_ref[...] + p1_ref[...], 0.0)
    h = h_ref[...] + lax.dot_general(
        agg, w_ref[...], (((1,), (1,)), ((), ())),
        preferred_element_type=jnp.float32) + b_ref[...]
    mean = jnp.mean(h, axis=0, keepdims=True)
    d = h - mean
    var = jnp.mean(d * d, axis=0, keepdims=True)
    hn = d * lax.rsqrt(var + 1e-5) * g_ref[...] + be_ref[...]
    out_refs[0][...] = hn
    if not last:
        key = jnp.exp(jnp.sum(hn * ak_ref[...], axis=1, keepdims=True)
                      * INV_SQRT_H)
        out_refs[1][...] = hn * key


_f32 = lambda *shape: jax.ShapeDtypeStruct(shape, jnp.float32)

_proj_call = pl.pallas_call(_proj_body, out_shape=[_f32(N, H), _f32(N, H)])
_mid_call = pl.pallas_call(functools.partial(_layer_body, False),
                           out_shape=[_f32(N, H), _f32(N, H)])
_last_call = pl.pallas_call(functools.partial(_layer_body, True),
                            out_shape=[_f32(N, H)])


def kernel(node_init, edge_index, edge_len, W_in, b_in, W1, b1, W2, b2,
           g1, be1, g2, be2, att_k):
    pad = EPAD - E
    srcp = jnp.pad(edge_index[0], (0, pad)).reshape(-1, CHUNK)
    dstp = jnp.pad(edge_index[1], (0, pad)).reshape(-1, CHUNK)
    pk = jnp.pad(jnp.stack([srcp, dstp], axis=1).reshape(-1), (0, PK))
    lenf = jnp.pad(edge_len, (0, pad + CHUNK), constant_values=np.inf)
    zeros = jnp.zeros((ROWS_PS, H), jnp.float32)
    b_in2 = b_in.reshape(1, H)
    b1_2, b2_2 = b1.reshape(1, H), b2.reshape(1, H)
    g1_2, g2_2 = g1.reshape(1, H), g2.reshape(1, H)
    be1_2, be2_2 = be1.reshape(1, H), be2.reshape(1, H)
    ak2 = att_k.reshape(1, H)

    h0, m0 = _proj_call(node_init, W_in, b_in2, ak2)
    part = _edge_kernel(m0, pk, lenf, zeros)
    h1, m1 = _mid_call(h0, part[0, :N], part[1, :N], W1, b1_2, g1_2,
                       be1_2, ak2)
    part = _edge_kernel(m1, pk, lenf, zeros)
    h2 = _last_call(h1, part[0, :N], part[1, :N], W2, b2_2, g2_2, be2_2,
                    ak2)
    return h2[0] if isinstance(h2, (list, tuple)) else h2


# speedup vs baseline: 1.1933x; 1.0209x over previous
import functools

import jax
import jax.numpy as jnp
import numpy as np
from jax import lax
from jax.experimental import pallas as pl
from jax.experimental.pallas import tpu as pltpu
from jax.experimental.pallas import tpu_sc as plsc

N, F, H, E = 10000, 128, 128, 320000
NC, NS = 2, 16
NW = NC * NS
CHUNK = 80
K0 = 162
K1 = 90
PK = 2 * CHUNK
EPAD = NS * (K0 + K1) * CHUNK
NP = 10240
ROWS_PS = NP // NS
LANES = 16
INV_SQRT_H = float(1.0 / np.sqrt(H))


_GDN = lax.GatherDimensionNumbers(offset_dims=(), collapsed_slice_dims=(0,),
                                  start_index_map=(0,))


def _bcast_lane(vec, lane):
    idx = jnp.full((LANES, 1), lane, jnp.int32)
    return lax.gather(vec, idx, _GDN, slice_sizes=(1,),
                      mode=lax.GatherScatterMode.PROMISE_IN_BOUNDS)

def _edge_body(m_hbm, pk_hbm, len_hbm, zeros_hbm, out_hbm,
               agg, i0, i1, w0, w1, d0, d1, g0, g1, s0, s1,
               im0, im1, wm0, wm1, gm0, gm1, sm0, sm1):
    ibufs, wbufs, dbufs = (i0, i1), (w0, w1), (d0, d1)
    gbufs, sbufs = (g0, g1), (s0, s1)
    isems, wsems = (im0, im1), (wm0, wm1)
    gsems, ssems = (gm0, gm1), (sm0, sm1)
    c = lax.axis_index("c")
    sid = lax.axis_index("s")
    base = jnp.where(c == 0, sid * K0, NS * K0 + sid * K1)
    nch = jnp.where(c == 0, K0, K1)

    def idx_cp(j, b):
        return pltpu.make_async_copy(
            pk_hbm.at[pl.ds((base + j) * PK, PK)], ibufs[b], isems[b])

    def len_cp(j, b):
        return pltpu.make_async_copy(
            len_hbm.at[pl.ds((base + j) * CHUNK, CHUNK)], wbufs[b],
            wsems[b])

    def gather_cp(b):
        return pltpu.make_async_copy(
            m_hbm.at[ibufs[b].at[pl.ds(0, CHUNK)]], gbufs[b], gsems[b])

    def scatter_issue(b):
        pltpu.async_copy(sbufs[b], agg.at[dbufs[b].at[0]], ssems[b],
                         add=True)

    def scatter_wait(b):
        pltpu.make_async_copy(sbufs[b], agg.at[dbufs[b].at[0]],
                              ssems[b]).wait()

    def copy_dst(b):
        for q in range(CHUNK // LANES):
            sl = pl.ds(q * LANES, LANES)
            dbufs[b][0, sl] = ibufs[b][pl.ds(CHUNK + q * LANES, LANES)]

    def scale(b):
        def _group(gi, inner):
            wvec = 1.0 / (1.0 + wbufs[b][pl.ds(gi * LANES, LANES)])
            for l in range(LANES):
                wb = _bcast_lane(wvec, l)
                e = gi * LANES + l
                for r in range(H // LANES):
                    sl = pl.ds(r * LANES, LANES)
                    sbufs[b][e, sl] = gbufs[b][e, sl] * wb
            return inner

        lax.fori_loop(0, CHUNK // LANES, _group, 0)

    pltpu.sync_copy(zeros_hbm, agg.at[pl.ds(sid * ROWS_PS, ROWS_PS)])
    idx_cp(0, 0).start()
    len_cp(0, 0).start()
    idx_cp(1, 1).start()
    len_cp(1, 1).start()
    idx_cp(0, 0).wait()
    len_cp(0, 0).wait()
    gather_cp(0).start()
    plsc.subcore_barrier()

    def step(j, b, first, last):
        if not last:
            idx_cp(j + 1, 1 - b).wait()
            len_cp(j + 1, 1 - b).wait()
            gather_cp(1 - b).start()
        gather_cp(b).wait()
        if not first:
            scatter_wait(b)
        copy_dst(b)
        scale(b)
        scatter_issue(b)
        if not (first or last):
            idx_cp(j + 2, b).start()
            len_cp(j + 2, b).start()

    step(0, 0, True, False)
    idx_cp(2, 0).start()
    len_cp(2, 0).start()
    step(1, 1, True, False)
    idx_cp(3, 1).start()
    len_cp(3, 1).start()

    def _steady(o, carry):
        step(2 * o, 0, False, False)
        step(2 * o + 1, 1, False, False)
        return carry

    lax.fori_loop(1, nch // 2 - 1, _steady, 0)

    step(nch - 2, 0, False, False)
    step(nch - 1, 1, False, True)
    scatter_wait(0)
    scatter_wait(1)
    idx_cp(nch, 0).wait()
    len_cp(nch, 0).wait()

    plsc.subcore_barrier()

    pltpu.sync_copy(agg.at[pl.ds(sid * ROWS_PS, ROWS_PS)],
                    out_hbm.at[c, pl.ds(sid * ROWS_PS, ROWS_PS)])


@functools.partial(
    pl.kernel,
    out_type=jax.ShapeDtypeStruct((NC, NP, H), jnp.float32),
    mesh=plsc.VectorSubcoreMesh(core_axis_name="c", subcore_axis_name="s",
                                num_cores=NC, num_subcores=NS),
    scratch_types=[
        pltpu.VMEM_SHARED((NP, H), jnp.float32),
        pltpu.VMEM((PK,), jnp.int32),
        pltpu.VMEM((PK,), jnp.int32),
        pltpu.VMEM((CHUNK,), jnp.float32),
        pltpu.VMEM((CHUNK,), jnp.float32),
        pltpu.VMEM((1, CHUNK), jnp.int32),
        pltpu.VMEM((1, CHUNK), jnp.int32),
        pltpu.VMEM((CHUNK, H), jnp.float32),
        pltpu.VMEM((CHUNK, H), jnp.float32),
        pltpu.VMEM((CHUNK, H), jnp.float32),
        pltpu.VMEM((CHUNK, H), jnp.float32),
    ] + [pltpu.SemaphoreType.DMA] * 8,
)
def _edge_kernel(*args):
    _edge_body(*args)


def _proj_body(x_ref, w_ref, b_ref, ak_ref, h_ref, m_ref):
    x = x_ref[...]
    h = lax.dot_general(x, w_ref[...], (((1,), (1,)), ((), ())),
                        preferred_element_type=jnp.float32) + b_ref[...]
    h_ref[...] = h
    key = jnp.exp(jnp.sum(h * ak_ref[...], axis=1, keepdims=True)
                  * INV_SQRT_H)
    m_ref[...] = h * key


def _layer_body(last, h_ref, p0_ref, p1_ref, w_ref, b_ref, g_ref, be_ref,
                ak_ref, *out_refs):
    agg = jnp.maximum(p0_ref[...] + p1_ref[...], 0.0)
    h = h_ref[...] + lax.dot_general(
        agg, w_ref[...], (((1,), (1,)), ((), ())),
        preferred_element_type=jnp.float32) + b_ref[...]
    mean = jnp.mean(h, axis=0, keepdims=True)
    d = h - mean
    var = jnp.mean(d * d, axis=0, keepdims=True)
    hn = d * lax.rsqrt(var + 1e-5) * g_ref[...] + be_ref[...]
    out_refs[0][...] = hn
    if not last:
        key = jnp.exp(jnp.sum(hn * ak_ref[...], axis=1, keepdims=True)
                      * INV_SQRT_H)
        out_refs[1][...] = hn * key


_f32 = lambda *shape: jax.ShapeDtypeStruct(shape, jnp.float32)

_proj_call = pl.pallas_call(_proj_body, out_shape=[_f32(N, H), _f32(N, H)])
_mid_call = pl.pallas_call(functools.partial(_layer_body, False),
                           out_shape=[_f32(N, H), _f32(N, H)])
_last_call = pl.pallas_call(functools.partial(_layer_body, True),
                            out_shape=[_f32(N, H)])


def kernel(node_init, edge_index, edge_len, W_in, b_in, W1, b1, W2, b2,
           g1, be1, g2, be2, att_k):
    pad = EPAD - E
    srcp = jnp.pad(edge_index[0], (0, pad)).reshape(-1, CHUNK)
    dstp = jnp.pad(edge_index[1], (0, pad)).reshape(-1, CHUNK)
    pk = jnp.pad(jnp.stack([srcp, dstp], axis=1).reshape(-1), (0, PK))
    lenf = jnp.pad(edge_len, (0, pad + CHUNK), constant_values=np.inf)
    zeros = jnp.zeros((ROWS_PS, H), jnp.float32)
    b_in2 = b_in.reshape(1, H)
    b1_2, b2_2 = b1.reshape(1, H), b2.reshape(1, H)
    g1_2, g2_2 = g1.reshape(1, H), g2.reshape(1, H)
    be1_2, be2_2 = be1.reshape(1, H), be2.reshape(1, H)
    ak2 = att_k.reshape(1, H)

    h0, m0 = _proj_call(node_init, W_in, b_in2, ak2)
    part = _edge_kernel(m0, pk, lenf, zeros)
    h1, m1 = _mid_call(h0, part[0, :N], part[1, :N], W1, b1_2, g1_2,
                       be1_2, ak2)
    part = _edge_kernel(m1, pk, lenf, zeros)
    h2 = _last_call(h1, part[0, :N], part[1, :N], W2, b2_2, g2_2, be2_2,
                    ak2)
    return h2[0] if isinstance(h2, (list, tuple)) else h2
